# baseline (speedup 1.0000x reference)
import functools

import jax
import jax.numpy as jnp
from jax.experimental import pallas as pl

_K = 32
_SX = 2
_SY = 2
_CHUNK = 500


def _mm_body(x_ref, w_ref, o_ref):
    o_ref[...] = jnp.dot(x_ref[...], w_ref[...],
                         preferred_element_type=jnp.float32)


def _node_transform(x, wr, block_rows):
    n, d = x.shape
    f = wr.shape[1]
    assert n % block_rows == 0
    grid = (n // block_rows,)
    return pl.pallas_call(
        _mm_body,
        grid=grid,
        in_specs=[
            pl.BlockSpec((block_rows, d), lambda i: (i, 0)),
            pl.BlockSpec((d, f), lambda i: (0, 0)),
        ],
        out_specs=pl.BlockSpec((block_rows, f), lambda i: (i, 0)),
        out_shape=jax.ShapeDtypeStruct((n, f), jnp.float32),
    )(x, wr)


def _rbf_basis(x, size):
    centers = jnp.linspace(-1.0, 1.0, size)
    sigma = 2.0 / max(size - 1, 1)
    return jnp.exp(-((x[:, None] - centers[None, :]) ** 2) / (2.0 * sigma ** 2))


def _radius_edges(positions, output, support):
    n = positions.shape[0]
    def chunk_fn(pos_chunk):
        d2 = jnp.sum((pos_chunk[:, None, :] - output[None, :, :]) ** 2, axis=-1)
        neg, idx = jax.lax.top_k(-d2, _K)
        return neg, idx
    pc = positions.reshape(n // _CHUNK, _CHUNK, 2)
    neg, idx = jax.lax.map(chunk_fn, pc)
    neg = neg.reshape(n * _K)
    idx = idx.reshape(n * _K)
    row = jnp.repeat(jnp.arange(n, dtype=jnp.int32), _K)
    col = idx.astype(jnp.int32)
    mask = (-neg) <= support * support
    return row, col, mask


def kernel(positions, features, output, ghostIndices, support, W1, W2):
    n = positions.shape[0]
    m = output.shape[0]
    row, col, emask = _radius_edges(positions, output, support)
    pseudo = output[col] - positions[row]
    pseudo = pseudo / support
    pseudo = jnp.clip(pseudo, -1.0, 1.0)
    bx = _rbf_basis(pseudo[:, 0], _SX)
    by = _rbf_basis(pseudo[:, 1], _SY)
    ghosts = ghostIndices[m:]
    em = emask.astype(jnp.float32)[:, None]
    ans = features
    for W in (W1, W2):
        d = W.shape[2]
        f = W.shape[3]
        wr = jnp.transpose(W, (2, 0, 1, 3)).reshape(d, _SX * _SY * f)
        t = _node_transform(ans, wr, block_rows=1000)
        msg = jnp.zeros((row.shape[0], f), dtype=jnp.float32)
        for u in range(_SX):
            for v in range(_SY):
                coeff = (bx[:, u] * by[:, v])[:, None]
                sl = (u * _SY + v) * f
                msg = msg + coeff * jnp.take(t[:, sl:sl + f], row, axis=0)
        msg = msg * em
        aggr = jax.ops.segment_sum(msg, col, num_segments=m)
        ansc = jax.nn.relu(aggr)
        ghostFeatures = jnp.take(ansc, ghosts, axis=0)
        ans = jnp.concatenate([ansc, ghostFeatures], axis=0)
    return ans


# speedup vs baseline: 10.4151x; 10.4151x over previous
import functools

import jax
import jax.numpy as jnp
from jax import lax
from jax.experimental import pallas as pl
from jax.experimental.pallas import tpu as pltpu
from jax.experimental.pallas import tpu_sc as plsc

_K = 32
_SX = 2
_SY = 2
_NP = 10240
_MP = 9088
_MPP = _MP + 16
_QPW = _NP // 32
_CAP = 144
_BUFN = 176
_EB = 128
_INF = 1e30


def _mesh():
    return plsc.VectorSubcoreMesh(core_axis_name="c", subcore_axis_name="s",
                                  num_cores=2, num_subcores=16)


def _spi(x):
    return jnp.zeros((16,), jnp.int32) + x


def _spf(x):
    return jnp.zeros((16,), jnp.float32) + x


def _select_body(sox_h, soy_h, ord_h, qx_h, qy_h, par_h,
                 idx_h, cnt_h, c_h,
                 sox, soy, ordv, qx, qy, par,
                 d2b, posb, selb, idx_st, c_st, cnt_st):
    cid = lax.axis_index("c")
    sid = lax.axis_index("s")
    wid = sid * 2 + cid
    pltpu.sync_copy(sox_h, sox)
    pltpu.sync_copy(soy_h, soy)
    pltpu.sync_copy(ord_h, ordv)
    pltpu.sync_copy(qx_h.at[pl.ds(wid * _QPW, _QPW)], qx)
    pltpu.sync_copy(qy_h.at[pl.ds(wid * _QPW, _QPW)], qy)
    pltpu.sync_copy(par_h, par)

    iota16 = lax.iota(jnp.int32, 16)
    lane0 = iota16 == 0
    zero16 = jnp.zeros((16,), jnp.float32)
    inf16 = jnp.full((16,), _INF, jnp.float32)
    parv = par[...]
    r2s = parv[0]
    rs = parv[1]
    invrv = _spf(parv[2])
    r2v = _spf(r2s)

    def per_block(qb, _carry):
        def per_query(ql, _c2):
            i = qb * 16 + ql
            qxv = plsc.load_gather(qx, [_spi(i)])
            qyv = plsc.load_gather(qy, [_spi(i)])
            qxi = qxv[0]
            qlo = qxi - rs
            qhi = qxi + rs

            def bs(_t, lh):
                lo, hi = lh
                mid = (lo + hi) >> 1
                p = sox[pl.ds(mid, 16)][0] < qlo
                return (jnp.where(p, mid + 1, lo), jnp.where(p, hi, mid))
            lo, _ = lax.fori_loop(0, 14, bs, (jnp.int32(0), jnp.int32(_MP)))
            base = (lo >> 4) << 4

            def sc_cond(st):
                return st[2]

            def sc_body(st):
                pos, cnt, _go = st
                ox = sox[pl.ds(pos, 16)]
                oy = soy[pl.ds(pos, 16)]
                dx = ox - qxv
                dy = oy - qyv
                d2 = dx * dx + dy * dy
                pred = d2 <= r2v
                plsc.store_compressed(posb.at[pl.ds(cnt, 16)], pos + iota16,
                                      mask=pred)
                plsc.store_compressed(d2b.at[pl.ds(cnt, 16)], d2, mask=pred)
                cnt2 = cnt + jnp.sum(pred.astype(jnp.int32))
                pos2 = pos + 16
                go2 = ((pos2 < _MP) & (jnp.max(ox) <= qhi)
                       & (cnt2 <= _CAP))
                return (pos2, cnt2, go2)

            _, cnt, _ = lax.while_loop(
                sc_cond, sc_body,
                (base, jnp.int32(0), base < _MP))

            d2b[pl.ds(cnt, 16)] = inf16
            nsel = jnp.minimum(cnt, _K)
            nch = (cnt + 15) >> 4

            def ext(k, _c3):
                def ch(ci, best):
                    bd, bp = best
                    d2c = d2b[pl.ds(ci * 16, 16)]
                    mn = jnp.min(d2c)
                    pin = jnp.min(jnp.where(d2c == mn, iota16, 16))
                    better = mn < bd
                    return (jnp.where(better, mn, bd),
                            jnp.where(better, ci * 16 + pin, bp))
                _, bpos = lax.fori_loop(0, nch, ch,
                                        (jnp.float32(_INF), jnp.int32(0)))
                pv = plsc.load_gather(posb, [_spi(bpos)])
                plsc.store_scatter(selb, [_spi(k)], pv, mask=lane0)
                plsc.store_scatter(d2b, [_spi(bpos)], inf16, mask=lane0)
                return _c3
            lax.fori_loop(0, nsel, ext, 0)

            for kk in range(2):
                lanes = iota16 + kk * 16
                msk = lanes < nsel
                sp = selb[pl.ds(kk * 16, 16)]
                ox = plsc.load_gather(sox, [sp], mask=msk)
                oy = plsc.load_gather(soy, [sp], mask=msk)
                jt = plsc.load_gather(ordv, [sp], mask=msk)
                px = jnp.clip((ox - qxv) * invrv, -1.0, 1.0)
                py = jnp.clip((oy - qyv) * invrv, -1.0, 1.0)
                ex0 = jnp.exp((px + 1.0) * (px + 1.0) * -0.125)
                ex1 = jnp.exp((px - 1.0) * (px - 1.0) * -0.125)
                ey0 = jnp.exp((py + 1.0) * (py + 1.0) * -0.125)
                ey1 = jnp.exp((py - 1.0) * (py - 1.0) * -0.125)
                idx_st[ql, pl.ds(kk * 16, 16)] = jnp.where(msk, jt, 0)
                c_st[ql, 0, pl.ds(kk * 16, 16)] = jnp.where(msk, ex0 * ey0,
                                                            zero16)
                c_st[ql, 1, pl.ds(kk * 16, 16)] = jnp.where(msk, ex0 * ey1,
                                                            zero16)
                c_st[ql, 2, pl.ds(kk * 16, 16)] = jnp.where(msk, ex1 * ey0,
                                                            zero16)
                c_st[ql, 3, pl.ds(kk * 16, 16)] = jnp.where(msk, ex1 * ey1,
                                                            zero16)
            plsc.store_scatter(cnt_st, [_spi(ql)], _spi(nsel), mask=lane0)
            return _c2
        lax.fori_loop(0, 16, per_query, 0)
        qg = wid * _QPW + qb * 16
        pltpu.sync_copy(idx_st, idx_h.at[pl.ds(qg, 16)])
        pltpu.sync_copy(c_st, c_h.at[pl.ds(qg, 16)])
        pltpu.sync_copy(cnt_st, cnt_h.at[pl.ds(qg, 16)])
        return _carry
    lax.fori_loop(0, _QPW // 16, per_block, 0)


def _select_call(sox, soy, ordp, qx, qy, par):
    return pl.kernel(
        _select_body,
        out_type=[
            jax.ShapeDtypeStruct((_NP, _K), jnp.int32),
            jax.ShapeDtypeStruct((_NP,), jnp.int32),
            jax.ShapeDtypeStruct((_NP, 4, _K), jnp.float32),
        ],
        mesh=_mesh(),
        scratch_types=[
            pltpu.VMEM((_MPP,), jnp.float32),
            pltpu.VMEM((_MPP,), jnp.float32),
            pltpu.VMEM((_MPP,), jnp.int32),
            pltpu.VMEM((_QPW,), jnp.float32),
            pltpu.VMEM((_QPW,), jnp.float32),
            pltpu.VMEM((16,), jnp.float32),
            pltpu.VMEM((_BUFN,), jnp.float32),
            pltpu.VMEM((_BUFN,), jnp.int32),
            pltpu.VMEM((_K,), jnp.int32),
            pltpu.VMEM((16, _K), jnp.int32),
            pltpu.VMEM((16, 4, _K), jnp.float32),
            pltpu.VMEM((16,), jnp.int32),
        ],
        compiler_params=pltpu.CompilerParams(needs_layout_passes=False),
    )(sox, soy, ordp, qx, qy, par)


def _accum_body(t_h, idx_h, c_h, cnt_h,
                out_h,
                tblk, cblk, idxblk, cntblk, ebuf, eidx, acc):
    cid = lax.axis_index("c")
    sid = lax.axis_index("s")
    wid = sid * 2 + cid
    rows_per_tile = _MP // 16
    iota16 = lax.iota(jnp.int32, 16)
    lane0 = iota16 == 0
    zero16 = jnp.zeros((16,), jnp.float32)
    zero16i = jnp.zeros((16,), jnp.int32)

    def zrow(ri, _c):
        for j in range(8):
            ebuf[ri, pl.ds(j * 16, 16)] = zero16
        return _c
    lax.fori_loop(0, _EB, zrow, 0)
    for b in range(4):
        pltpu.sync_copy(ebuf, acc.at[pl.ds(sid * rows_per_tile + b * _EB,
                                           _EB)])
    pltpu.sync_copy(ebuf.at[pl.ds(0, rows_per_tile - 4 * _EB)],
                    acc.at[pl.ds(sid * rows_per_tile + 4 * _EB,
                                 rows_per_tile - 4 * _EB)])
    plsc.subcore_barrier()

    def blk(qb, eptr):
        qg = wid * _QPW + qb * 32
        pltpu.sync_copy(t_h.at[pl.ds(qg, 32)], tblk)
        pltpu.sync_copy(c_h.at[pl.ds(qg, 32)], cblk)
        pltpu.sync_copy(idx_h.at[pl.ds(qg, 32)], idxblk)
        pltpu.sync_copy(cnt_h.at[pl.ds(qg, 32)], cntblk)

        def pq(ql, eptr):
            nc = plsc.load_gather(cntblk, [_spi(ql)])[0]
            qlv = _spi(ql)

            def pe(k, eptr):
                kv = _spi(k)
                c00 = plsc.load_gather(cblk, [qlv, zero16i, kv])
                c01 = plsc.load_gather(cblk, [qlv, zero16i + 1, kv])
                c10 = plsc.load_gather(cblk, [qlv, zero16i + 2, kv])
                c11 = plsc.load_gather(cblk, [qlv, zero16i + 3, kv])
                for rr in range(8):
                    t0 = tblk[ql, pl.ds(rr * 16, 16)]
                    t1 = tblk[ql, pl.ds(128 + rr * 16, 16)]
                    t2 = tblk[ql, pl.ds(256 + rr * 16, 16)]
                    t3 = tblk[ql, pl.ds(384 + rr * 16, 16)]
                    ms = t0 * c00 + t1 * c01 + t2 * c10 + t3 * c11
                    ebuf[eptr, pl.ds(rr * 16, 16)] = ms
                jtv = plsc.load_gather(idxblk, [qlv, kv])
                plsc.store_scatter(eidx, [_spi(eptr)], jtv, mask=lane0)
                eptr2 = eptr + 1
                full = eptr2 == _EB

                @pl.when(full)
                def _():
                    pltpu.sync_copy(ebuf, acc.at[eidx], add=True)
                return jnp.where(full, 0, eptr2)
            return lax.fori_loop(0, nc, pe, eptr)
        return lax.fori_loop(0, 32, pq, eptr)
    eptr = lax.fori_loop(0, _QPW // 32, blk, jnp.int32(0))

    def ztail(ri, _c):
        @pl.when(ri >= eptr)
        def _():
            for j in range(8):
                ebuf[ri, pl.ds(j * 16, 16)] = zero16
            plsc.store_scatter(eidx, [_spi(ri)], zero16i, mask=lane0)
        return _c
    lax.fori_loop(0, _EB, ztail, 0)

    @pl.when(eptr > 0)
    def _():
        pltpu.sync_copy(ebuf, acc.at[eidx], add=True)
    plsc.subcore_barrier()

    base = sid * rows_per_tile
    pltpu.sync_copy(acc.at[pl.ds(base, rows_per_tile)],
                    out_h.at[cid, pl.ds(base, rows_per_tile)])


def _accum_call(t, idx, c4, cnt):
    return pl.kernel(
        _accum_body,
        out_type=jax.ShapeDtypeStruct((2, _MP, 128), jnp.float32),
        mesh=_mesh(),
        scratch_types=[
            pltpu.VMEM((32, 4 * 128), jnp.float32),
            pltpu.VMEM((32, 4, _K), jnp.float32),
            pltpu.VMEM((32, _K), jnp.int32),
            pltpu.VMEM((32,), jnp.int32),
            pltpu.VMEM((_EB, 128), jnp.float32),
            pltpu.VMEM((_EB,), jnp.int32),
            pltpu.VMEM_SHARED((_MP, 128), jnp.float32),
        ],
        compiler_params=pltpu.CompilerParams(needs_layout_passes=False),
    )(t, idx, c4, cnt)


def _mm_body(x_ref, w_ref, o_ref):
    o_ref[...] = jnp.dot(x_ref[...], w_ref[...],
                         preferred_element_type=jnp.float32)


def _node_transform(x, wr):
    n, d = x.shape
    f = wr.shape[1]
    br = 640
    return pl.pallas_call(
        _mm_body,
        grid=(n // br,),
        in_specs=[
            pl.BlockSpec((br, d), lambda i: (i, 0)),
            pl.BlockSpec((d, f), lambda i: (0, 0)),
        ],
        out_specs=pl.BlockSpec((br, f), lambda i: (i, 0)),
        out_shape=jax.ShapeDtypeStruct((n, f), jnp.float32),
    )(x, wr)


def _comb_body(p_ref, o_ref):
    o_ref[...] = jnp.maximum(p_ref[0] + p_ref[1], 0.0)


def _combine(parts):
    br = 1128
    return pl.pallas_call(
        _comb_body,
        grid=(_MP // br,),
        in_specs=[pl.BlockSpec((2, br, 128), lambda i: (0, i, 0))],
        out_specs=pl.BlockSpec((br, 128), lambda i: (i, 0)),
        out_shape=jax.ShapeDtypeStruct((_MP, 128), jnp.float32),
    )(parts)


def kernel(positions, features, output, ghostIndices, support, W1, W2):
    n = positions.shape[0]
    m = output.shape[0]
    d = features.shape[1]
    sup = jnp.float32(support)
    par = jnp.zeros((16,), jnp.float32)
    par = par.at[0].set(sup * sup).at[1].set(sup).at[2].set(1.0 / sup)

    ox = output[:, 0]
    order = jnp.argsort(ox).astype(jnp.int32)
    pad_m = jnp.full((_MPP - m,), 2.0, jnp.float32)
    sox = jnp.concatenate([ox[order], pad_m])
    soy = jnp.concatenate([output[:, 1][order], pad_m])
    ordp = jnp.concatenate([order, jnp.zeros((_MPP - m,), jnp.int32)])
    pad_n = jnp.full((_NP - n,), 2.0, jnp.float32)
    qx = jnp.concatenate([positions[:, 0], pad_n])
    qy = jnp.concatenate([positions[:, 1], pad_n])

    idxA, cntA, cA = _select_call(sox, soy, ordp, qx, qy, par)

    ghosts = ghostIndices[m:]
    ans = features
    for W in (W1, W2):
        wr = jnp.transpose(W, (2, 0, 1, 3)).reshape(d, _SX * _SY * d)
        xpad = jnp.concatenate(
            [ans, jnp.zeros((_NP - n, d), jnp.float32)], axis=0)
        t = _node_transform(xpad, wr)
        parts = _accum_call(t, idxA, cA, cntA)
        ansc = _combine(parts)[:m]
        ans = jnp.concatenate([ansc, jnp.take(ansc, ghosts, axis=0)], axis=0)
    return ans


# speedup vs baseline: 16.5851x; 1.5924x over previous
import functools

import jax
import jax.numpy as jnp
from jax import lax
from jax.experimental import pallas as pl
from jax.experimental.pallas import tpu as pltpu
from jax.experimental.pallas import tpu_sc as plsc

_K = 32
_SX = 2
_SY = 2
_NP = 10240
_MP = 9088
_MPP = _MP + 16
_QPW = _NP // 32
_CAP = 144
_BUFN = 176
_EB = 128
_INF = 1e30


def _mesh():
    return plsc.VectorSubcoreMesh(core_axis_name="c", subcore_axis_name="s",
                                  num_cores=2, num_subcores=16)


def _spi(x):
    return jnp.zeros((16,), jnp.int32) + x


def _spf(x):
    return jnp.zeros((16,), jnp.float32) + x


def _select_body(sox_h, soy_h, ord_h, qx_h, qy_h, par_h,
                 idx_h, cnt_h, c_h,
                 sox, soy, ordv, qx, qy, par,
                 d2b, posb, selb, idx_st, c_st, cnt_st):
    cid = lax.axis_index("c")
    sid = lax.axis_index("s")
    wid = sid * 2 + cid
    pltpu.sync_copy(sox_h, sox)
    pltpu.sync_copy(soy_h, soy)
    pltpu.sync_copy(ord_h, ordv)
    pltpu.sync_copy(qx_h.at[pl.ds(wid * _QPW, _QPW)], qx)
    pltpu.sync_copy(qy_h.at[pl.ds(wid * _QPW, _QPW)], qy)
    pltpu.sync_copy(par_h, par)

    iota16 = lax.iota(jnp.int32, 16)
    lane0 = iota16 == 0
    zero16 = jnp.zeros((16,), jnp.float32)
    inf16 = jnp.full((16,), _INF, jnp.float32)
    parv = par[...]
    r2s = parv[0]
    rs = parv[1]
    invrv = _spf(parv[2])
    r2v = _spf(r2s)

    def per_block(qb, _carry):
        def per_query(ql, _c2):
            i = qb * 16 + ql
            qxv = plsc.load_gather(qx, [_spi(i)])
            qyv = plsc.load_gather(qy, [_spi(i)])
            qxi = qxv[0]
            qlo = qxi - rs
            qhi = qxi + rs

            def bs(_t, lh):
                lo, hi = lh
                mid = (lo + hi) >> 1
                p = sox[pl.ds(mid, 16)][0] < qlo
                return (jnp.where(p, mid + 1, lo), jnp.where(p, hi, mid))
            lo, _ = lax.fori_loop(0, 14, bs, (jnp.int32(0), jnp.int32(_MP)))
            base = (lo >> 4) << 4

            def sc_cond(st):
                return st[2]

            def sc_body(st):
                pos, cnt, _go = st
                ox = sox[pl.ds(pos, 16)]
                oy = soy[pl.ds(pos, 16)]
                dx = ox - qxv
                dy = oy - qyv
                d2 = dx * dx + dy * dy
                pred = d2 <= r2v
                plsc.store_compressed(posb.at[pl.ds(cnt, 16)], pos + iota16,
                                      mask=pred)
                plsc.store_compressed(d2b.at[pl.ds(cnt, 16)], d2, mask=pred)
                cnt2 = cnt + plsc.all_reduce_population_count(pred)[0]
                pos2 = pos + 16
                go2 = ((pos2 < _MP) & (ox[15] <= qhi)
                       & (cnt2 <= _CAP))
                return (pos2, cnt2, go2)

            _, cnt, _ = lax.while_loop(
                sc_cond, sc_body,
                (base, jnp.int32(0), base < _MP))

            d2b[pl.ds(cnt, 16)] = inf16
            nsel = jnp.minimum(cnt, _K)
            nch = (cnt + 15) >> 4

            @pl.when(cnt > _K)
            def _():
                def ext(k, _c3):
                    def ch(ci, best):
                        bd, bp = best
                        d2c = d2b[pl.ds(ci * 16, 16)]
                        mn = jnp.min(d2c)
                        pin = jnp.min(jnp.where(d2c == mn, iota16, 16))
                        better = mn < bd
                        return (jnp.where(better, mn, bd),
                                jnp.where(better, ci * 16 + pin, bp))
                    _, bpos = lax.fori_loop(0, nch, ch,
                                            (jnp.float32(_INF), jnp.int32(0)))
                    pv = plsc.load_gather(posb, [_spi(bpos)])
                    plsc.store_scatter(selb, [_spi(k)], pv, mask=lane0)
                    plsc.store_scatter(d2b, [_spi(bpos)], inf16, mask=lane0)
                    return _c3
                lax.fori_loop(0, _K, ext, 0)
                posb[pl.ds(0, 16)] = selb[pl.ds(0, 16)]
                posb[pl.ds(16, 16)] = selb[pl.ds(16, 16)]

            for kk in range(2):
                lanes = iota16 + kk * 16
                msk = lanes < nsel
                sp = posb[pl.ds(kk * 16, 16)]
                ox = plsc.load_gather(sox, [sp], mask=msk)
                oy = plsc.load_gather(soy, [sp], mask=msk)
                jt = plsc.load_gather(ordv, [sp], mask=msk)
                px = jnp.clip((ox - qxv) * invrv, -1.0, 1.0)
                py = jnp.clip((oy - qyv) * invrv, -1.0, 1.0)
                ex0 = jnp.exp((px + 1.0) * (px + 1.0) * -0.125)
                ex1 = jnp.exp((px - 1.0) * (px - 1.0) * -0.125)
                ey0 = jnp.exp((py + 1.0) * (py + 1.0) * -0.125)
                ey1 = jnp.exp((py - 1.0) * (py - 1.0) * -0.125)
                idx_st[ql, pl.ds(kk * 16, 16)] = jnp.where(msk, jt, 0)
                c_st[ql, 0, pl.ds(kk * 16, 16)] = jnp.where(msk, ex0 * ey0,
                                                            zero16)
                c_st[ql, 1, pl.ds(kk * 16, 16)] = jnp.where(msk, ex0 * ey1,
                                                            zero16)
                c_st[ql, 2, pl.ds(kk * 16, 16)] = jnp.where(msk, ex1 * ey0,
                                                            zero16)
                c_st[ql, 3, pl.ds(kk * 16, 16)] = jnp.where(msk, ex1 * ey1,
                                                            zero16)
            plsc.store_scatter(cnt_st, [_spi(ql)], _spi(nsel), mask=lane0)
            return _c2
        lax.fori_loop(0, 16, per_query, 0)
        qg = wid * _QPW + qb * 16
        pltpu.sync_copy(idx_st, idx_h.at[pl.ds(qg, 16)])
        pltpu.sync_copy(c_st, c_h.at[pl.ds(qg, 16)])
        pltpu.sync_copy(cnt_st, cnt_h.at[pl.ds(qg, 16)])
        return _carry
    lax.fori_loop(0, _QPW // 16, per_block, 0)


def _select_call(sox, soy, ordp, qx, qy, par):
    return pl.kernel(
        _select_body,
        out_type=[
            jax.ShapeDtypeStruct((_NP, _K), jnp.int32),
            jax.ShapeDtypeStruct((_NP,), jnp.int32),
            jax.ShapeDtypeStruct((_NP, 4, _K), jnp.float32),
        ],
        mesh=_mesh(),
        scratch_types=[
            pltpu.VMEM((_MPP,), jnp.float32),
            pltpu.VMEM((_MPP,), jnp.float32),
            pltpu.VMEM((_MPP,), jnp.int32),
            pltpu.VMEM((_QPW,), jnp.float32),
            pltpu.VMEM((_QPW,), jnp.float32),
            pltpu.VMEM((16,), jnp.float32),
            pltpu.VMEM((_BUFN,), jnp.float32),
            pltpu.VMEM((_BUFN,), jnp.int32),
            pltpu.VMEM((_K,), jnp.int32),
            pltpu.VMEM((16, _K), jnp.int32),
            pltpu.VMEM((16, 4, _K), jnp.float32),
            pltpu.VMEM((16,), jnp.int32),
        ],
        compiler_params=pltpu.CompilerParams(needs_layout_passes=False),
    )(sox, soy, ordp, qx, qy, par)


def _accum_body(t_h, idx_h, c_h, cnt_h,
                out_h,
                tblk, cblk, idxblk, cntblk, ebuf, eidx, acc):
    cid = lax.axis_index("c")
    sid = lax.axis_index("s")
    wid = sid * 2 + cid
    rows_per_tile = _MP // 16
    iota16 = lax.iota(jnp.int32, 16)
    lane0 = iota16 == 0
    zero16 = jnp.zeros((16,), jnp.float32)
    zero16i = jnp.zeros((16,), jnp.int32)

    def zrow(ri, _c):
        for j in range(8):
            ebuf[ri, pl.ds(j * 16, 16)] = zero16
        return _c
    lax.fori_loop(0, _EB, zrow, 0)
    for b in range(4):
        pltpu.sync_copy(ebuf, acc.at[pl.ds(sid * rows_per_tile + b * _EB,
                                           _EB)])
    pltpu.sync_copy(ebuf.at[pl.ds(0, rows_per_tile - 4 * _EB)],
                    acc.at[pl.ds(sid * rows_per_tile + 4 * _EB,
                                 rows_per_tile - 4 * _EB)])
    plsc.subcore_barrier()

    def blk(qb, eptr):
        qg = wid * _QPW + qb * 32
        pltpu.sync_copy(t_h.at[pl.ds(qg, 32)], tblk)
        pltpu.sync_copy(c_h.at[pl.ds(qg, 32)], cblk)
        pltpu.sync_copy(idx_h.at[pl.ds(qg, 32)], idxblk)
        pltpu.sync_copy(cnt_h.at[pl.ds(qg, 32)], cntblk)

        def pq(ql, eptr):
            nc = plsc.load_gather(cntblk, [_spi(ql)])[0]
            qlv = _spi(ql)
            trow = [tblk[ql, pl.ds(q * 16, 16)] for q in range(32)]

            def pe(k, eptr):
                kv = _spi(k)
                c00 = plsc.load_gather(cblk, [qlv, zero16i, kv])
                c01 = plsc.load_gather(cblk, [qlv, zero16i + 1, kv])
                c10 = plsc.load_gather(cblk, [qlv, zero16i + 2, kv])
                c11 = plsc.load_gather(cblk, [qlv, zero16i + 3, kv])
                for rr in range(8):
                    ms = (trow[rr] * c00 + trow[8 + rr] * c01
                          + trow[16 + rr] * c10 + trow[24 + rr] * c11)
                    ebuf[eptr, pl.ds(rr * 16, 16)] = ms
                jtv = plsc.load_gather(idxblk, [qlv, kv])
                plsc.store_scatter(eidx, [_spi(eptr)], jtv, mask=lane0)
                eptr2 = eptr + 1
                full = eptr2 == _EB

                @pl.when(full)
                def _():
                    pltpu.sync_copy(ebuf, acc.at[eidx], add=True)
                return jnp.where(full, 0, eptr2)
            return lax.fori_loop(0, nc, pe, eptr)
        return lax.fori_loop(0, 32, pq, eptr)
    eptr = lax.fori_loop(0, _QPW // 32, blk, jnp.int32(0))

    def ztail(ri, _c):
        @pl.when(ri >= eptr)
        def _():
            for j in range(8):
                ebuf[ri, pl.ds(j * 16, 16)] = zero16
            plsc.store_scatter(eidx, [_spi(ri)], zero16i, mask=lane0)
        return _c
    lax.fori_loop(0, _EB, ztail, 0)

    @pl.when(eptr > 0)
    def _():
        pltpu.sync_copy(ebuf, acc.at[eidx], add=True)
    plsc.subcore_barrier()

    base = sid * rows_per_tile
    pltpu.sync_copy(acc.at[pl.ds(base, rows_per_tile)],
                    out_h.at[cid, pl.ds(base, rows_per_tile)])


def _accum_call(t, idx, c4, cnt):
    return pl.kernel(
        _accum_body,
        out_type=jax.ShapeDtypeStruct((2, _MP, 128), jnp.float32),
        mesh=_mesh(),
        scratch_types=[
            pltpu.VMEM((32, 4 * 128), jnp.float32),
            pltpu.VMEM((32, 4, _K), jnp.float32),
            pltpu.VMEM((32, _K), jnp.int32),
            pltpu.VMEM((32,), jnp.int32),
            pltpu.VMEM((_EB, 128), jnp.float32),
            pltpu.VMEM((_EB,), jnp.int32),
            pltpu.VMEM_SHARED((_MP, 128), jnp.float32),
        ],
        compiler_params=pltpu.CompilerParams(needs_layout_passes=False),
    )(t, idx, c4, cnt)


def _mm_body(x_ref, w_ref, o_ref):
    o_ref[...] = jnp.dot(x_ref[...], w_ref[...],
                         preferred_element_type=jnp.float32)


def _node_transform(x, wr):
    n, d = x.shape
    f = wr.shape[1]
    br = 640
    return pl.pallas_call(
        _mm_body,
        grid=(n // br,),
        in_specs=[
            pl.BlockSpec((br, d), lambda i: (i, 0)),
            pl.BlockSpec((d, f), lambda i: (0, 0)),
        ],
        out_specs=pl.BlockSpec((br, f), lambda i: (i, 0)),
        out_shape=jax.ShapeDtypeStruct((n, f), jnp.float32),
    )(x, wr)


def _comb_body(p_ref, o_ref):
    o_ref[...] = jnp.maximum(p_ref[0] + p_ref[1], 0.0)


def _combine(parts):
    br = 1128
    return pl.pallas_call(
        _comb_body,
        grid=(_MP // br,),
        in_specs=[pl.BlockSpec((2, br, 128), lambda i: (0, i, 0))],
        out_specs=pl.BlockSpec((br, 128), lambda i: (i, 0)),
        out_shape=jax.ShapeDtypeStruct((_MP, 128), jnp.float32),
    )(parts)


def kernel(positions, features, output, ghostIndices, support, W1, W2):
    n = positions.shape[0]
    m = output.shape[0]
    d = features.shape[1]
    sup = jnp.float32(support)
    par = jnp.zeros((16,), jnp.float32)
    par = par.at[0].set(sup * sup).at[1].set(sup).at[2].set(1.0 / sup)

    ox = output[:, 0]
    order = jnp.argsort(ox).astype(jnp.int32)
    pad_m = jnp.full((_MPP - m,), 2.0, jnp.float32)
    sox = jnp.concatenate([ox[order], pad_m])
    soy = jnp.concatenate([output[:, 1][order], pad_m])
    ordp = jnp.concatenate([order, jnp.zeros((_MPP - m,), jnp.int32)])
    pad_n = jnp.full((_NP - n,), 2.0, jnp.float32)
    qx = jnp.concatenate([positions[:, 0], pad_n])
    qy = jnp.concatenate([positions[:, 1], pad_n])

    idxA, cntA, cA = _select_call(sox, soy, ordp, qx, qy, par)

    ghosts = ghostIndices[m:]
    ans = features
    for W in (W1, W2):
        wr = jnp.transpose(W, (2, 0, 1, 3)).reshape(d, _SX * _SY * d)
        xpad = jnp.concatenate(
            [ans, jnp.zeros((_NP - n, d), jnp.float32)], axis=0)
        t = _node_transform(xpad, wr)
        parts = _accum_call(t, idxA, cA, cntA)
        ansc = _combine(parts)[:m]
        ans = jnp.concatenate([ansc, jnp.take(ansc, ghosts, axis=0)], axis=0)
    return ans


# speedup vs baseline: 16.8370x; 1.0152x over previous
import functools

import jax
import jax.numpy as jnp
from jax import lax
from jax.experimental import pallas as pl
from jax.experimental.pallas import tpu as pltpu
from jax.experimental.pallas import tpu_sc as plsc

_K = 32
_SX = 2
_SY = 2
_NP = 10240
_MP = 9088
_MPP = _MP + 16
_QPW = _NP // 32
_CAP = 144
_BUFN = 176
_EB = 128
_INF = 1e30


def _mesh():
    return plsc.VectorSubcoreMesh(core_axis_name="c", subcore_axis_name="s",
                                  num_cores=2, num_subcores=16)


def _spi(x):
    return jnp.zeros((16,), jnp.int32) + x


def _spf(x):
    return jnp.zeros((16,), jnp.float32) + x


def _select_body(sox_h, soy_h, ord_h, qx_h, qy_h, par_h,
                 idx_h, cnt_h, c_h,
                 sox, soy, ordv, qx, qy, par,
                 d2b, posb, selb, idx_st, c_st, cnt_st):
    cid = lax.axis_index("c")
    sid = lax.axis_index("s")
    wid = sid * 2 + cid
    pltpu.sync_copy(sox_h, sox)
    pltpu.sync_copy(soy_h, soy)
    pltpu.sync_copy(ord_h, ordv)
    pltpu.sync_copy(qx_h.at[pl.ds(wid * _QPW, _QPW)], qx)
    pltpu.sync_copy(qy_h.at[pl.ds(wid * _QPW, _QPW)], qy)
    pltpu.sync_copy(par_h, par)

    iota16 = lax.iota(jnp.int32, 16)
    lane0 = iota16 == 0
    zero16 = jnp.zeros((16,), jnp.float32)
    inf16 = jnp.full((16,), _INF, jnp.float32)
    parv = par[...]
    r2s = parv[0]
    rs = parv[1]
    invrv = _spf(parv[2])
    r2v = _spf(r2s)

    def per_query(i, _c2):
            qxv = plsc.load_gather(qx, [_spi(i)])
            qyv = plsc.load_gather(qy, [_spi(i)])
            qxi = qxv[0]
            qlo = qxi - rs
            qhi = qxi + rs

            def bs(_t, lh):
                lo, hi = lh
                mid = (lo + hi) >> 1
                p = sox[pl.ds(mid, 16)][0] < qlo
                return (jnp.where(p, mid + 1, lo), jnp.where(p, hi, mid))
            lo, _ = lax.fori_loop(0, 14, bs, (jnp.int32(0), jnp.int32(_MP)))
            base = (lo >> 4) << 4

            def sc_cond(st):
                return st[2]

            def sc_body(st):
                pos, cnt, _go = st
                ox = sox[pl.ds(pos, 16)]
                oy = soy[pl.ds(pos, 16)]
                dx = ox - qxv
                dy = oy - qyv
                d2 = dx * dx + dy * dy
                pred = d2 <= r2v
                plsc.store_compressed(posb.at[pl.ds(cnt, 16)], pos + iota16,
                                      mask=pred)
                plsc.store_compressed(d2b.at[pl.ds(cnt, 16)], d2, mask=pred)
                cnt2 = cnt + plsc.all_reduce_population_count(pred)[0]
                pos2 = pos + 16
                go2 = ((pos2 < _MP) & (ox[15] <= qhi)
                       & (cnt2 <= _CAP))
                return (pos2, cnt2, go2)

            _, cnt, _ = lax.while_loop(
                sc_cond, sc_body,
                (base, jnp.int32(0), base < _MP))

            d2b[pl.ds(cnt, 16)] = inf16
            nsel = jnp.minimum(cnt, _K)
            nch = (cnt + 15) >> 4

            @pl.when(cnt > _K)
            def _():
                def ext(k, _c3):
                    def ch(ci, best):
                        bd, bp = best
                        d2c = d2b[pl.ds(ci * 16, 16)]
                        mn = jnp.min(d2c)
                        pin = jnp.min(jnp.where(d2c == mn, iota16, 16))
                        better = mn < bd
                        return (jnp.where(better, mn, bd),
                                jnp.where(better, ci * 16 + pin, bp))
                    _, bpos = lax.fori_loop(0, nch, ch,
                                            (jnp.float32(_INF), jnp.int32(0)))
                    pv = plsc.load_gather(posb, [_spi(bpos)])
                    plsc.store_scatter(selb, [_spi(k)], pv, mask=lane0)
                    plsc.store_scatter(d2b, [_spi(bpos)], inf16, mask=lane0)
                    return _c3
                lax.fori_loop(0, _K, ext, 0)
                posb[pl.ds(0, 16)] = selb[pl.ds(0, 16)]
                posb[pl.ds(16, 16)] = selb[pl.ds(16, 16)]

            for kk in range(2):
                lanes = iota16 + kk * 16
                msk = lanes < nsel
                sp = posb[pl.ds(kk * 16, 16)]
                ox = plsc.load_gather(sox, [sp], mask=msk)
                oy = plsc.load_gather(soy, [sp], mask=msk)
                jt = plsc.load_gather(ordv, [sp], mask=msk)
                px = jnp.clip((ox - qxv) * invrv, -1.0, 1.0)
                py = jnp.clip((oy - qyv) * invrv, -1.0, 1.0)
                ex0 = jnp.exp((px + 1.0) * (px + 1.0) * -0.125)
                ex1 = jnp.exp((px - 1.0) * (px - 1.0) * -0.125)
                ey0 = jnp.exp((py + 1.0) * (py + 1.0) * -0.125)
                ey1 = jnp.exp((py - 1.0) * (py - 1.0) * -0.125)
                idx_st[pl.ds(i * _K + kk * 16, 16)] = jnp.where(msk, jt, 0)
                cb = i * 4 * _K + kk * 16
                c_st[pl.ds(cb, 16)] = jnp.where(msk, ex0 * ey0, zero16)
                c_st[pl.ds(cb + _K, 16)] = jnp.where(msk, ex0 * ey1, zero16)
                c_st[pl.ds(cb + 2 * _K, 16)] = jnp.where(msk, ex1 * ey0,
                                                         zero16)
                c_st[pl.ds(cb + 3 * _K, 16)] = jnp.where(msk, ex1 * ey1,
                                                         zero16)
            plsc.store_scatter(cnt_st, [_spi(i)], _spi(nsel), mask=lane0)
            return _c2
    lax.fori_loop(0, _QPW, per_query, 0)
    qg = wid * _QPW
    pltpu.sync_copy(idx_st, idx_h.at[pl.ds(qg * _K, _QPW * _K)])
    pltpu.sync_copy(c_st, c_h.at[pl.ds(qg * 4 * _K, _QPW * 4 * _K)])
    pltpu.sync_copy(cnt_st, cnt_h.at[pl.ds(qg, _QPW)])


def _select_call(sox, soy, ordp, qx, qy, par):
    return pl.kernel(
        _select_body,
        out_type=[
            jax.ShapeDtypeStruct((_NP * _K,), jnp.int32),
            jax.ShapeDtypeStruct((_NP,), jnp.int32),
            jax.ShapeDtypeStruct((_NP * 4 * _K,), jnp.float32),
        ],
        mesh=_mesh(),
        scratch_types=[
            pltpu.VMEM((_MPP,), jnp.float32),
            pltpu.VMEM((_MPP,), jnp.float32),
            pltpu.VMEM((_MPP,), jnp.int32),
            pltpu.VMEM((_QPW,), jnp.float32),
            pltpu.VMEM((_QPW,), jnp.float32),
            pltpu.VMEM((16,), jnp.float32),
            pltpu.VMEM((_BUFN,), jnp.float32),
            pltpu.VMEM((_BUFN,), jnp.int32),
            pltpu.VMEM((_K,), jnp.int32),
            pltpu.VMEM((_QPW * _K,), jnp.int32),
            pltpu.VMEM((_QPW * 4 * _K,), jnp.float32),
            pltpu.VMEM((_QPW,), jnp.int32),
        ],
        compiler_params=pltpu.CompilerParams(needs_layout_passes=False),
    )(sox, soy, ordp, qx, qy, par)


def _accum_body(t_h, idx_h, c_h, cnt_h,
                out_h,
                tblk, cblk, idxblk, cntblk, ebuf, eidx, acc):
    cid = lax.axis_index("c")
    sid = lax.axis_index("s")
    wid = sid * 2 + cid
    rows_per_tile = _MP // 16
    iota16 = lax.iota(jnp.int32, 16)
    lane0 = iota16 == 0
    zero16 = jnp.zeros((16,), jnp.float32)
    zero16i = jnp.zeros((16,), jnp.int32)

    def zrow(ri, _c):
        for j in range(8):
            ebuf[ri, pl.ds(j * 16, 16)] = zero16
        return _c
    lax.fori_loop(0, _EB, zrow, 0)
    for b in range(4):
        pltpu.sync_copy(ebuf, acc.at[pl.ds(sid * rows_per_tile + b * _EB,
                                           _EB)])
    pltpu.sync_copy(ebuf.at[pl.ds(0, rows_per_tile - 4 * _EB)],
                    acc.at[pl.ds(sid * rows_per_tile + 4 * _EB,
                                 rows_per_tile - 4 * _EB)])
    plsc.subcore_barrier()

    qg0 = wid * _QPW

    def blk(qb, eptr):
        qg = qg0 + qb * 32
        pltpu.sync_copy(t_h.at[pl.ds(qg, 32)], tblk)
        pltpu.sync_copy(c_h.at[pl.ds(qg * 4 * _K, 32 * 4 * _K)], cblk)
        pltpu.sync_copy(idx_h.at[pl.ds(qg * _K, 32 * _K)], idxblk)
        pltpu.sync_copy(cnt_h.at[pl.ds(qg, 32)], cntblk)

        def pq(ql, eptr):
            nc = plsc.load_gather(cntblk, [_spi(ql)])[0]
            qlv = _spi(ql)
            trow = [tblk[ql, pl.ds(q * 16, 16)] for q in range(32)]

            def pe(k, eptr):
                cbase = qlv * (4 * _K) + k
                c00 = plsc.load_gather(cblk, [cbase])
                c01 = plsc.load_gather(cblk, [cbase + _K])
                c10 = plsc.load_gather(cblk, [cbase + 2 * _K])
                c11 = plsc.load_gather(cblk, [cbase + 3 * _K])
                for rr in range(8):
                    ms = (trow[rr] * c00 + trow[8 + rr] * c01
                          + trow[16 + rr] * c10 + trow[24 + rr] * c11)
                    ebuf[eptr, pl.ds(rr * 16, 16)] = ms
                jtv = plsc.load_gather(idxblk, [qlv * _K + k])
                plsc.store_scatter(eidx, [_spi(eptr)], jtv, mask=lane0)
                eptr2 = eptr + 1
                full = eptr2 == _EB

                @pl.when(full)
                def _():
                    pltpu.sync_copy(ebuf, acc.at[eidx], add=True)
                return jnp.where(full, 0, eptr2)
            return lax.fori_loop(0, nc, pe, eptr)
        return lax.fori_loop(0, 32, pq, eptr)
    eptr = lax.fori_loop(0, _QPW // 32, blk, jnp.int32(0))

    def ztail(ri, _c):
        @pl.when(ri >= eptr)
        def _():
            for j in range(8):
                ebuf[ri, pl.ds(j * 16, 16)] = zero16
            plsc.store_scatter(eidx, [_spi(ri)], zero16i, mask=lane0)
        return _c
    lax.fori_loop(0, _EB, ztail, 0)

    @pl.when(eptr > 0)
    def _():
        pltpu.sync_copy(ebuf, acc.at[eidx], add=True)
    plsc.subcore_barrier()

    base = sid * rows_per_tile
    pltpu.sync_copy(acc.at[pl.ds(base, rows_per_tile)],
                    out_h.at[cid, pl.ds(base, rows_per_tile)])


def _accum_call(t, idx, c4, cnt):
    return pl.kernel(
        _accum_body,
        out_type=jax.ShapeDtypeStruct((2, _MP, 128), jnp.float32),
        mesh=_mesh(),
        scratch_types=[
            pltpu.VMEM((32, 4 * 128), jnp.float32),
            pltpu.VMEM((32 * 4 * _K,), jnp.float32),
            pltpu.VMEM((32 * _K,), jnp.int32),
            pltpu.VMEM((32,), jnp.int32),
            pltpu.VMEM((_EB, 128), jnp.float32),
            pltpu.VMEM((_EB,), jnp.int32),
            pltpu.VMEM_SHARED((_MP, 128), jnp.float32),
        ],
        compiler_params=pltpu.CompilerParams(needs_layout_passes=False),
    )(t, idx, c4, cnt)


def _mm_body(x_ref, w_ref, o_ref):
    o_ref[...] = jnp.dot(x_ref[...], w_ref[...],
                         preferred_element_type=jnp.float32)


def _node_transform(x, wr):
    n, d = x.shape
    f = wr.shape[1]
    br = 640
    return pl.pallas_call(
        _mm_body,
        grid=(n // br,),
        in_specs=[
            pl.BlockSpec((br, d), lambda i: (i, 0)),
            pl.BlockSpec((d, f), lambda i: (0, 0)),
        ],
        out_specs=pl.BlockSpec((br, f), lambda i: (i, 0)),
        out_shape=jax.ShapeDtypeStruct((n, f), jnp.float32),
    )(x, wr)


def _comb_body(p_ref, o_ref):
    o_ref[...] = jnp.maximum(p_ref[0] + p_ref[1], 0.0)


def _combine(parts):
    br = 1128
    return pl.pallas_call(
        _comb_body,
        grid=(_MP // br,),
        in_specs=[pl.BlockSpec((2, br, 128), lambda i: (0, i, 0))],
        out_specs=pl.BlockSpec((br, 128), lambda i: (i, 0)),
        out_shape=jax.ShapeDtypeStruct((_MP, 128), jnp.float32),
    )(parts)


def kernel(positions, features, output, ghostIndices, support, W1, W2):
    n = positions.shape[0]
    m = output.shape[0]
    d = features.shape[1]
    sup = jnp.float32(support)
    par = jnp.zeros((16,), jnp.float32)
    par = par.at[0].set(sup * sup).at[1].set(sup).at[2].set(1.0 / sup)

    ox = output[:, 0]
    order = jnp.argsort(ox).astype(jnp.int32)
    pad_m = jnp.full((_MPP - m,), 2.0, jnp.float32)
    sox = jnp.concatenate([ox[order], pad_m])
    soy = jnp.concatenate([output[:, 1][order], pad_m])
    ordp = jnp.concatenate([order, jnp.zeros((_MPP - m,), jnp.int32)])
    pad_n = jnp.full((_NP - n,), 2.0, jnp.float32)
    qx = jnp.concatenate([positions[:, 0], pad_n])
    qy = jnp.concatenate([positions[:, 1], pad_n])

    idxA, cntA, cA = _select_call(sox, soy, ordp, qx, qy, par)

    ghosts = ghostIndices[m:]
    ans = features
    for W in (W1, W2):
        wr = jnp.transpose(W, (2, 0, 1, 3)).reshape(d, _SX * _SY * d)
        xpad = jnp.concatenate(
            [ans, jnp.zeros((_NP - n, d), jnp.float32)], axis=0)
        t = _node_transform(xpad, wr)
        parts = _accum_call(t, idxA, cA, cntA)
        ansc = _combine(parts)[:m]
        ans = jnp.concatenate([ansc, jnp.take(ansc, ghosts, axis=0)], axis=0)
    return ans
